# Initial kernel scaffold; baseline (speedup 1.0000x reference)
import jax
import jax.numpy as jnp
from jax.experimental import pallas as pl


def kernel(x, edge_index, edge_attr, batch, W1, b1, W2, b2, Wc, bc):
    raise NotImplementedError("write your pallas kernel here")


# speedup vs baseline: 6.6943x; 6.6943x over previous
import functools

import jax
import jax.numpy as jnp
from jax import lax
from jax.experimental import pallas as pl
from jax.experimental.pallas import tpu as pltpu
from jax.experimental.pallas import tpu_sc as plsc

_NW = 32
_NS = 16
_CH = 80
_BR = 1024
_G = 64


def _deg_kernel(dst, ewx, npad):
    e = dst.shape[0]
    per = e // _NW
    nch = per // _CH
    rpt = npad // _NS

    mesh = plsc.VectorSubcoreMesh(core_axis_name="c", subcore_axis_name="s")

    @functools.partial(
        pl.kernel,
        mesh=mesh,
        out_type=jax.ShapeDtypeStruct((2, npad, 128), jnp.float32),
        scratch_types=[
            pltpu.VMEM((_CH,), jnp.int32),
            pltpu.VMEM((_CH, 16), jnp.float32),
            pltpu.VMEM((_CH, 128), jnp.float32),
            pltpu.VMEM_SHARED((npad, 128), jnp.float32),
        ],
    )
    def k(dst_h, ewx_h, out_h, dst_v, ew_v, rows, acc):
        c = lax.axis_index("c")
        s = lax.axis_index("s")
        wid = c * _NS + s
        base = wid * per
        zero16 = jnp.zeros((16,), jnp.float32)

        def zrow(r, _):
            for j in range(8):
                rows[r, pl.ds(16 * j, 16)] = zero16
            return 0

        lax.fori_loop(0, _CH, zrow, 0)
        for t in range(rpt // _CH):
            pltpu.sync_copy(rows, acc.at[pl.ds(s * rpt + t * _CH, _CH)])
        plsc.subcore_barrier()

        def chunk(kk, _):
            st = pl.multiple_of(base + kk * _CH, 8)
            pltpu.sync_copy(dst_h.at[pl.ds(st, _CH)], dst_v)
            pltpu.sync_copy(ewx_h.at[pl.ds(st, _CH)], ew_v)

            def srow(i, _):
                bc = ew_v[i, :]
                for j in range(8):
                    rows[i, pl.ds(16 * j, 16)] = bc
                return 0

            lax.fori_loop(0, _CH, srow, 0)
            pltpu.sync_copy(rows, acc.at[dst_v], add=True)
            return 0

        lax.fori_loop(0, nch, chunk, 0)
        plsc.subcore_barrier()
        pltpu.sync_copy(acc.at[pl.ds(s * rpt, rpt)],
                        out_h.at[c, pl.ds(s * rpt, rpt)])

    return k(dst, ewx)


def _agg_kernel(hs, src, dst, ewx, npad):
    e = src.shape[0]
    per = e // _NW
    nch = per // _CH
    rpt = npad // _NS

    mesh = plsc.VectorSubcoreMesh(core_axis_name="c", subcore_axis_name="s")

    @functools.partial(
        pl.kernel,
        mesh=mesh,
        out_type=jax.ShapeDtypeStruct((2, npad, 128), jnp.float32),
        scratch_types=[
            pltpu.VMEM((_CH,), jnp.int32),
            pltpu.VMEM((_CH,), jnp.int32),
            pltpu.VMEM((_CH, 16), jnp.float32),
            pltpu.VMEM((_CH, 128), jnp.float32),
            pltpu.VMEM_SHARED((npad, 128), jnp.float32),
            pltpu.SemaphoreType.DMA,
        ],
    )
    def k(hs_h, src_h, dst_h, ewx_h, out_h, src_v, dst_v, ew_v, rows, acc,
          sem):
        c = lax.axis_index("c")
        s = lax.axis_index("s")
        wid = c * _NS + s
        base = wid * per
        zero16 = jnp.zeros((16,), jnp.float32)

        def zrow(r, _):
            for j in range(8):
                rows[r, pl.ds(16 * j, 16)] = zero16
            return 0

        lax.fori_loop(0, _CH, zrow, 0)
        for t in range(rpt // _CH):
            pltpu.sync_copy(rows, acc.at[pl.ds(s * rpt + t * _CH, _CH)])
        plsc.subcore_barrier()

        def chunk(kk, _):
            st = pl.multiple_of(base + kk * _CH, 8)
            pltpu.sync_copy(src_h.at[pl.ds(st, _CH)], src_v)
            pltpu.sync_copy(dst_h.at[pl.ds(st, _CH)], dst_v)
            pltpu.sync_copy(ewx_h.at[pl.ds(st, _CH)], ew_v)
            pltpu.async_copy(hs_h.at[src_v], rows, sem).wait()

            def srow(i, _):
                bc = ew_v[i, :]
                for j in range(8):
                    rows[i, pl.ds(16 * j, 16)] = rows[i, pl.ds(16 * j, 16)] * bc
                return 0

            lax.fori_loop(0, _CH, srow, 0)
            pltpu.sync_copy(rows, acc.at[dst_v], add=True)
            return 0

        lax.fori_loop(0, nch, chunk, 0)
        plsc.subcore_barrier()
        pltpu.sync_copy(acc.at[pl.ds(s * rpt, rpt)],
                        out_h.at[c, pl.ds(s * rpt, rpt)])

    return k(hs, src, dst, ewx)


def _tc_stage1(x_p, deg0, deg1, w1):
    npad = x_p.shape[0]
    grid = (npad // _BR,)

    def body(xb, d0, d1, w, hb, hsb, dvb):
        dv = lax.rsqrt(d0[...] + d1[...] + 1.0)
        h = jnp.dot(xb[...], w[...], preferred_element_type=jnp.float32)
        hb[...] = h
        hsb[...] = h * dv
        dvb[...] = dv

    return pl.pallas_call(
        body,
        grid=grid,
        in_specs=[
            pl.BlockSpec((_BR, 128), lambda i: (i, 0)),
            pl.BlockSpec((_BR, 1), lambda i: (i, 0)),
            pl.BlockSpec((_BR, 1), lambda i: (i, 0)),
            pl.BlockSpec((128, 128), lambda i: (0, 0)),
        ],
        out_specs=[
            pl.BlockSpec((_BR, 128), lambda i: (i, 0)),
            pl.BlockSpec((_BR, 128), lambda i: (i, 0)),
            pl.BlockSpec((_BR, 1), lambda i: (i, 0)),
        ],
        out_shape=[
            jax.ShapeDtypeStruct((npad, 128), jnp.float32),
            jax.ShapeDtypeStruct((npad, 128), jnp.float32),
            jax.ShapeDtypeStruct((npad, 1), jnp.float32),
        ],
    )(x_p, deg0, deg1, w1)


def _tc_stage2(a0, a1, h1, dinv, b1, w2):
    npad = h1.shape[0]
    grid = (npad // _BR,)

    def body(a0b, a1b, hb, dvb, bb, wb, h2o, hs2o):
        dv = dvb[...]
        u = jnp.maximum(
            dv * (a0b[...] + a1b[...]) + dv * dv * hb[...] + bb[...], 0.0)
        h2 = jnp.dot(u, wb[...], preferred_element_type=jnp.float32)
        h2o[...] = h2
        hs2o[...] = h2 * dv

    return pl.pallas_call(
        body,
        grid=grid,
        in_specs=[
            pl.BlockSpec((_BR, 128), lambda i: (i, 0)),
            pl.BlockSpec((_BR, 128), lambda i: (i, 0)),
            pl.BlockSpec((_BR, 128), lambda i: (i, 0)),
            pl.BlockSpec((_BR, 1), lambda i: (i, 0)),
            pl.BlockSpec((1, 128), lambda i: (0, 0)),
            pl.BlockSpec((128, 128), lambda i: (0, 0)),
        ],
        out_specs=[
            pl.BlockSpec((_BR, 128), lambda i: (i, 0)),
            pl.BlockSpec((_BR, 128), lambda i: (i, 0)),
        ],
        out_shape=[
            jax.ShapeDtypeStruct((npad, 128), jnp.float32),
            jax.ShapeDtypeStruct((npad, 128), jnp.float32),
        ],
    )(a0, a1, h1, dinv, b1, w2)


def _tc_stage3(a0, a1, h2, dinv, b2, batch_row, wc, bc):
    npad = h2.shape[0]
    grid = (npad // _BR,)
    nc = wc.shape[1]

    def body(a0b, a1b, hb, dvb, bb, btb, wcb, bcb, out, pacc):
        i = pl.program_id(0)
        dv = dvb[...]
        v = jnp.maximum(
            dv * (a0b[...] + a1b[...]) + dv * dv * hb[...] + bb[...], 0.0)
        mt = (btb[...] == lax.broadcasted_iota(jnp.int32, (_G, _BR), 0))
        pb = lax.dot_general(mt.astype(jnp.float32), v,
                             (((1,), (0,)), ((), ())),
                             preferred_element_type=jnp.float32)

        @pl.when(i == 0)
        def _():
            pacc[...] = pb

        @pl.when(i > 0)
        def _():
            pacc[...] = pacc[...] + pb

        @pl.when(i == pl.num_programs(0) - 1)
        def _():
            logits = jnp.dot(pacc[...], wcb[...],
                             preferred_element_type=jnp.float32) + bcb[...]
            out[...] = jax.nn.sigmoid(logits)

    return pl.pallas_call(
        body,
        grid=grid,
        in_specs=[
            pl.BlockSpec((_BR, 128), lambda i: (i, 0)),
            pl.BlockSpec((_BR, 128), lambda i: (i, 0)),
            pl.BlockSpec((_BR, 128), lambda i: (i, 0)),
            pl.BlockSpec((_BR, 1), lambda i: (i, 0)),
            pl.BlockSpec((1, 128), lambda i: (0, 0)),
            pl.BlockSpec((1, _BR), lambda i: (0, i)),
            pl.BlockSpec((128, nc), lambda i: (0, 0)),
            pl.BlockSpec((1, nc), lambda i: (0, 0)),
        ],
        out_specs=pl.BlockSpec((_G, nc), lambda i: (0, 0)),
        out_shape=jax.ShapeDtypeStruct((_G, nc), jnp.float32),
        scratch_shapes=[pltpu.VMEM((_G, 128), jnp.float32)],
    )(a0, a1, h2, dinv, b2, batch_row, wc, bc)


def kernel(x, edge_index, edge_attr, batch, W1, b1, W2, b2, Wc, bc):
    n, d = x.shape
    npad = ((n + _BR - 1) // _BR) * _BR

    x_p = jnp.pad(x, ((0, npad - n), (0, 0)))
    batch_row = jnp.pad(batch, (0, npad - n),
                        constant_values=_G).reshape(1, npad)
    src = edge_index[0]
    dst = edge_index[1]
    ewx = jnp.broadcast_to(edge_attr[:, None], (edge_attr.shape[0], 16))

    degs = _deg_kernel(dst, ewx, npad)
    deg0 = lax.slice(degs, (0, 0, 0), (1, npad, 1)).reshape(npad, 1)
    deg1 = lax.slice(degs, (1, 0, 0), (2, npad, 1)).reshape(npad, 1)

    h1, hs1, dinv = _tc_stage1(x_p, deg0, deg1, W1)

    agg1 = _agg_kernel(hs1, src, dst, ewx, npad)
    h2, hs2 = _tc_stage2(agg1[0], agg1[1], h1, dinv,
                         b1.reshape(1, -1), W2)

    agg2 = _agg_kernel(hs2, src, dst, ewx, npad)
    return _tc_stage3(agg2[0], agg2[1], h2, dinv,
                      b2.reshape(1, -1), batch_row, Wc, bc.reshape(1, -1))


# speedup vs baseline: 8.1944x; 1.2241x over previous
import functools

import jax
import jax.numpy as jnp
from jax import lax
from jax.experimental import pallas as pl
from jax.experimental.pallas import tpu as pltpu
from jax.experimental.pallas import tpu_sc as plsc

_NW = 32
_NS = 16
_CH = 80
_BR = 1024
_G = 64


def _deg_kernel(dst, ewx, npad):
    e = dst.shape[0]
    per = e // _NW
    nch = per // _CH
    rpt = npad // _NS

    mesh = plsc.VectorSubcoreMesh(core_axis_name="c", subcore_axis_name="s")

    @functools.partial(
        pl.kernel,
        mesh=mesh,
        out_type=jax.ShapeDtypeStruct((2, npad, 128), jnp.float32),
        scratch_types=[
            pltpu.VMEM((_CH,), jnp.int32),
            pltpu.VMEM((_CH, 16), jnp.float32),
            pltpu.VMEM((_CH, 128), jnp.float32),
            pltpu.VMEM_SHARED((npad, 128), jnp.float32),
        ],
    )
    def k(dst_h, ewx_h, out_h, dst_v, ew_v, rows, acc):
        c = lax.axis_index("c")
        s = lax.axis_index("s")
        wid = c * _NS + s
        base = wid * per
        zero16 = jnp.zeros((16,), jnp.float32)

        def zrow(r, _):
            for j in range(8):
                rows[r, pl.ds(16 * j, 16)] = zero16
            return 0

        lax.fori_loop(0, _CH, zrow, 0)
        for t in range(rpt // _CH):
            pltpu.sync_copy(rows, acc.at[pl.ds(s * rpt + t * _CH, _CH)])
        plsc.subcore_barrier()

        def chunk(kk, _):
            st = pl.multiple_of(base + kk * _CH, 8)
            pltpu.sync_copy(dst_h.at[pl.ds(st, _CH)], dst_v)
            pltpu.sync_copy(ewx_h.at[pl.ds(st, _CH)], ew_v)

            def srow(i, _):
                rows[i, pl.ds(0, 16)] = ew_v[i, :]
                return 0

            lax.fori_loop(0, _CH, srow, 0)
            pltpu.sync_copy(rows, acc.at[dst_v], add=True)
            return 0

        lax.fori_loop(0, nch, chunk, 0)
        plsc.subcore_barrier()
        pltpu.sync_copy(acc.at[pl.ds(s * rpt, rpt)],
                        out_h.at[c, pl.ds(s * rpt, rpt)])

    return k(dst, ewx)


def _agg_kernel(hs, src, dst, ewx, npad):
    e = src.shape[0]
    per = e // _NW
    nch = per // _CH
    rpt = npad // _NS

    mesh = plsc.VectorSubcoreMesh(core_axis_name="c", subcore_axis_name="s")

    @functools.partial(
        pl.kernel,
        mesh=mesh,
        out_type=jax.ShapeDtypeStruct((2, npad, 128), jnp.float32),
        scratch_types=[
            pltpu.VMEM((2, _CH), jnp.int32),
            pltpu.VMEM((2, _CH), jnp.int32),
            pltpu.VMEM((2, _CH, 16), jnp.float32),
            pltpu.VMEM((2, _CH, 128), jnp.float32),
            pltpu.VMEM_SHARED((npad, 128), jnp.float32),
            pltpu.SemaphoreType.DMA,
            pltpu.SemaphoreType.DMA,
        ],
    )
    def k(hs_h, src_h, dst_h, ewx_h, out_h, src_v, dst_v, ew_v, rows, acc,
          sem0, sem1):
        c = lax.axis_index("c")
        s = lax.axis_index("s")
        wid = c * _NS + s
        base = wid * per
        zero16 = jnp.zeros((16,), jnp.float32)
        sems = (sem0, sem1)

        def zrow(r, _):
            for j in range(8):
                rows[0, r, pl.ds(16 * j, 16)] = zero16
            return 0

        lax.fori_loop(0, _CH, zrow, 0)
        for t in range(rpt // _CH):
            pltpu.sync_copy(rows.at[0],
                            acc.at[pl.ds(s * rpt + t * _CH, _CH)])
        plsc.subcore_barrier()

        def prefetch(kk, b):
            st = pl.multiple_of(base + kk * _CH, 8)
            pltpu.sync_copy(src_h.at[pl.ds(st, _CH)], src_v.at[b])
            pltpu.sync_copy(dst_h.at[pl.ds(st, _CH)], dst_v.at[b])
            pltpu.sync_copy(ewx_h.at[pl.ds(st, _CH)], ew_v.at[b])
            pltpu.async_copy(hs_h.at[src_v.at[b]], rows.at[b], sems[b])

        def process(b):
            pltpu.make_async_copy(hs_h.at[src_v.at[b]], rows.at[b],
                                  sems[b]).wait()

            def srow(i, _):
                bc = ew_v[b, i, :]
                for j in range(8):
                    rows[b, i, pl.ds(16 * j, 16)] = (
                        rows[b, i, pl.ds(16 * j, 16)] * bc)
                return 0

            lax.fori_loop(0, _CH, srow, 0)
            pltpu.sync_copy(rows.at[b], acc.at[dst_v.at[b]], add=True)

        prefetch(0, 0)

        def pair(p, _):
            prefetch(2 * p + 1, 1)
            process(0)
            prefetch(2 * p + 2, 0)
            process(1)
            return 0

        lax.fori_loop(0, (nch - 1) // 2, pair, 0)
        process(0)
        plsc.subcore_barrier()
        pltpu.sync_copy(acc.at[pl.ds(s * rpt, rpt)],
                        out_h.at[c, pl.ds(s * rpt, rpt)])

    return k(hs, src, dst, ewx)


def _tc_stage1(x_p, deg0, deg1, w1):
    npad = x_p.shape[0]
    grid = (npad // _BR,)

    def body(xb, d0, d1, w, hb, hsb, dvb):
        dv = lax.rsqrt(d0[...] + d1[...] + 1.0)
        h = jnp.dot(xb[...], w[...], preferred_element_type=jnp.float32)
        hb[...] = h
        hsb[...] = h * dv
        dvb[...] = dv

    return pl.pallas_call(
        body,
        grid=grid,
        in_specs=[
            pl.BlockSpec((_BR, 128), lambda i: (i, 0)),
            pl.BlockSpec((_BR, 1), lambda i: (i, 0)),
            pl.BlockSpec((_BR, 1), lambda i: (i, 0)),
            pl.BlockSpec((128, 128), lambda i: (0, 0)),
        ],
        out_specs=[
            pl.BlockSpec((_BR, 128), lambda i: (i, 0)),
            pl.BlockSpec((_BR, 128), lambda i: (i, 0)),
            pl.BlockSpec((_BR, 1), lambda i: (i, 0)),
        ],
        out_shape=[
            jax.ShapeDtypeStruct((npad, 128), jnp.float32),
            jax.ShapeDtypeStruct((npad, 128), jnp.float32),
            jax.ShapeDtypeStruct((npad, 1), jnp.float32),
        ],
    )(x_p, deg0, deg1, w1)


def _tc_stage2(a0, a1, h1, dinv, b1, w2):
    npad = h1.shape[0]
    grid = (npad // _BR,)

    def body(a0b, a1b, hb, dvb, bb, wb, h2o, hs2o):
        dv = dvb[...]
        u = jnp.maximum(
            dv * (a0b[...] + a1b[...]) + dv * dv * hb[...] + bb[...], 0.0)
        h2 = jnp.dot(u, wb[...], preferred_element_type=jnp.float32)
        h2o[...] = h2
        hs2o[...] = h2 * dv

    return pl.pallas_call(
        body,
        grid=grid,
        in_specs=[
            pl.BlockSpec((_BR, 128), lambda i: (i, 0)),
            pl.BlockSpec((_BR, 128), lambda i: (i, 0)),
            pl.BlockSpec((_BR, 128), lambda i: (i, 0)),
            pl.BlockSpec((_BR, 1), lambda i: (i, 0)),
            pl.BlockSpec((1, 128), lambda i: (0, 0)),
            pl.BlockSpec((128, 128), lambda i: (0, 0)),
        ],
        out_specs=[
            pl.BlockSpec((_BR, 128), lambda i: (i, 0)),
            pl.BlockSpec((_BR, 128), lambda i: (i, 0)),
        ],
        out_shape=[
            jax.ShapeDtypeStruct((npad, 128), jnp.float32),
            jax.ShapeDtypeStruct((npad, 128), jnp.float32),
        ],
    )(a0, a1, h1, dinv, b1, w2)


def _tc_stage3(a0, a1, h2, dinv, b2, batch_row, wc, bc):
    npad = h2.shape[0]
    grid = (npad // _BR,)
    nc = wc.shape[1]

    def body(a0b, a1b, hb, dvb, bb, btb, wcb, bcb, out, pacc):
        i = pl.program_id(0)
        dv = dvb[...]
        v = jnp.maximum(
            dv * (a0b[...] + a1b[...]) + dv * dv * hb[...] + bb[...], 0.0)
        mt = (btb[...] == lax.broadcasted_iota(jnp.int32, (_G, _BR), 0))
        pb = lax.dot_general(mt.astype(jnp.float32), v,
                             (((1,), (0,)), ((), ())),
                             preferred_element_type=jnp.float32)

        @pl.when(i == 0)
        def _():
            pacc[...] = pb

        @pl.when(i > 0)
        def _():
            pacc[...] = pacc[...] + pb

        @pl.when(i == pl.num_programs(0) - 1)
        def _():
            logits = jnp.dot(pacc[...], wcb[...],
                             preferred_element_type=jnp.float32) + bcb[...]
            out[...] = jax.nn.sigmoid(logits)

    return pl.pallas_call(
        body,
        grid=grid,
        in_specs=[
            pl.BlockSpec((_BR, 128), lambda i: (i, 0)),
            pl.BlockSpec((_BR, 128), lambda i: (i, 0)),
            pl.BlockSpec((_BR, 128), lambda i: (i, 0)),
            pl.BlockSpec((_BR, 1), lambda i: (i, 0)),
            pl.BlockSpec((1, 128), lambda i: (0, 0)),
            pl.BlockSpec((1, _BR), lambda i: (0, i)),
            pl.BlockSpec((128, nc), lambda i: (0, 0)),
            pl.BlockSpec((1, nc), lambda i: (0, 0)),
        ],
        out_specs=pl.BlockSpec((_G, nc), lambda i: (0, 0)),
        out_shape=jax.ShapeDtypeStruct((_G, nc), jnp.float32),
        scratch_shapes=[pltpu.VMEM((_G, 128), jnp.float32)],
    )(a0, a1, h2, dinv, b2, batch_row, wc, bc)


def kernel(x, edge_index, edge_attr, batch, W1, b1, W2, b2, Wc, bc):
    n, d = x.shape
    npad = ((n + _BR - 1) // _BR) * _BR

    x_p = jnp.pad(x, ((0, npad - n), (0, 0)))
    batch_row = jnp.pad(batch, (0, npad - n),
                        constant_values=_G).reshape(1, npad)
    src = edge_index[0]
    dst = edge_index[1]
    ewx = jnp.broadcast_to(edge_attr[:, None], (edge_attr.shape[0], 16))

    degs = _deg_kernel(dst, ewx, npad)
    deg0 = lax.slice(degs, (0, 0, 0), (1, npad, 1)).reshape(npad, 1)
    deg1 = lax.slice(degs, (1, 0, 0), (2, npad, 1)).reshape(npad, 1)

    h1, hs1, dinv = _tc_stage1(x_p, deg0, deg1, W1)

    agg1 = _agg_kernel(hs1, src, dst, ewx, npad)
    h2, hs2 = _tc_stage2(agg1[0], agg1[1], h1, dinv,
                         b1.reshape(1, -1), W2)

    agg2 = _agg_kernel(hs2, src, dst, ewx, npad)
    return _tc_stage3(agg2[0], agg2[1], h2, dinv,
                      b2.reshape(1, -1), batch_row, Wc, bc.reshape(1, -1))


# speedup vs baseline: 9.1291x; 1.1141x over previous
import functools

import jax
import jax.numpy as jnp
from jax import lax
from jax.experimental import pallas as pl
from jax.experimental.pallas import tpu as pltpu
from jax.experimental.pallas import tpu_sc as plsc

_NW = 32
_NS = 16
_CH = 80
_BR = 1024
_G = 64


def _deg_kernel(sd, ewx, npad):
    e = ewx.shape[0]
    per = e // _NW
    nch = per // _CH
    rpt = npad // _NS

    mesh = plsc.VectorSubcoreMesh(core_axis_name="c", subcore_axis_name="s")

    @functools.partial(
        pl.kernel,
        mesh=mesh,
        out_type=jax.ShapeDtypeStruct((2, npad, 128), jnp.float32),
        scratch_types=[
            pltpu.VMEM((2, 2, _CH), jnp.int32),
            pltpu.VMEM((2, _CH, 16), jnp.float32),
            pltpu.VMEM((2, _CH, 128), jnp.float32),
            pltpu.VMEM_SHARED((npad, 128), jnp.float32),
            pltpu.SemaphoreType.DMA,
            pltpu.SemaphoreType.DMA,
        ],
    )
    def k(sd_h, ewx_h, out_h, sd_v, ew_v, rows, acc, ssem0, ssem1):
        c = lax.axis_index("c")
        s = lax.axis_index("s")
        wid = c * _NS + s
        base = wid * per
        cbase = wid * nch
        zero16 = jnp.zeros((16,), jnp.float32)
        ssems = (ssem0, ssem1)

        def zrow(r, _):
            for b in range(2):
                for j in range(8):
                    rows[b, r, pl.ds(16 * j, 16)] = zero16
            return 0

        lax.fori_loop(0, _CH, zrow, 0)
        for t in range(rpt // _CH):
            pltpu.sync_copy(rows.at[0],
                            acc.at[pl.ds(s * rpt + t * _CH, _CH)])
        plsc.subcore_barrier()

        def build(kk, b):
            st = pl.multiple_of(base + kk * _CH, 8)
            pltpu.sync_copy(sd_h.at[cbase + kk], sd_v.at[b])
            pltpu.sync_copy(ewx_h.at[pl.ds(st, _CH)], ew_v.at[b])

            def srow(i, _):
                rows[b, i, pl.ds(0, 16)] = ew_v[b, i, :]
                return 0

            lax.fori_loop(0, _CH, srow, 0)

        def scatter(b):
            pltpu.async_copy(rows.at[b], acc.at[sd_v.at[b, 1]], ssems[b],
                             add=True)

        def drain(b):
            pltpu.make_async_copy(rows.at[b], acc.at[sd_v.at[b, 1]],
                                  ssems[b]).wait()

        build(0, 0)
        scatter(0)

        def pair(p, _):
            build(2 * p + 1, 1)
            scatter(1)
            drain(0)
            build(2 * p + 2, 0)
            scatter(0)
            drain(1)
            return 0

        lax.fori_loop(0, (nch - 1) // 2, pair, 0)
        drain(0)
        plsc.subcore_barrier()
        pltpu.sync_copy(acc.at[pl.ds(s * rpt, rpt)],
                        out_h.at[c, pl.ds(s * rpt, rpt)])

    return k(sd, ewx)


def _agg_kernel(hs, sd, ewx, npad):
    e = ewx.shape[0]
    per = e // _NW
    nch = per // _CH
    rpt = npad // _NS

    mesh = plsc.VectorSubcoreMesh(core_axis_name="c", subcore_axis_name="s")

    @functools.partial(
        pl.kernel,
        mesh=mesh,
        out_type=jax.ShapeDtypeStruct((2, npad, 128), jnp.float32),
        scratch_types=[
            pltpu.VMEM((2, 2, _CH), jnp.int32),
            pltpu.VMEM((2, _CH, 16), jnp.float32),
            pltpu.VMEM((2, _CH, 128), jnp.float32),
            pltpu.VMEM_SHARED((npad, 128), jnp.float32),
            pltpu.SemaphoreType.DMA,
            pltpu.SemaphoreType.DMA,
            pltpu.SemaphoreType.DMA,
            pltpu.SemaphoreType.DMA,
        ],
    )
    def k(hs_h, sd_h, ewx_h, out_h, sd_v, ew_v, rows, acc,
          gsem0, gsem1, ssem0, ssem1):
        c = lax.axis_index("c")
        s = lax.axis_index("s")
        wid = c * _NS + s
        base = wid * per
        cbase = wid * nch
        zero16 = jnp.zeros((16,), jnp.float32)
        gsems = (gsem0, gsem1)
        ssems = (ssem0, ssem1)

        def zrow(r, _):
            for j in range(8):
                rows[0, r, pl.ds(16 * j, 16)] = zero16
            return 0

        lax.fori_loop(0, _CH, zrow, 0)
        for t in range(rpt // _CH):
            pltpu.sync_copy(rows.at[0],
                            acc.at[pl.ds(s * rpt + t * _CH, _CH)])
        plsc.subcore_barrier()

        def prefetch(kk, b):
            st = pl.multiple_of(base + kk * _CH, 8)
            pltpu.sync_copy(sd_h.at[cbase + kk], sd_v.at[b])
            pltpu.sync_copy(ewx_h.at[pl.ds(st, _CH)], ew_v.at[b])
            pltpu.async_copy(hs_h.at[sd_v.at[b, 0]], rows.at[b], gsems[b])

        def process(b):
            pltpu.make_async_copy(hs_h.at[sd_v.at[b, 0]], rows.at[b],
                                  gsems[b]).wait()

            def srow(i, _):
                bc = ew_v[b, i, :]
                for j in range(8):
                    rows[b, i, pl.ds(16 * j, 16)] = (
                        rows[b, i, pl.ds(16 * j, 16)] * bc)
                return 0

            lax.fori_loop(0, _CH, srow, 0)
            pltpu.async_copy(rows.at[b], acc.at[sd_v.at[b, 1]], ssems[b],
                             add=True)

        def drain(b):
            pltpu.make_async_copy(rows.at[b], acc.at[sd_v.at[b, 1]],
                                  ssems[b]).wait()

        prefetch(0, 0)
        prefetch(1, 1)
        process(0)

        def pair(p, _):
            drain(0)
            prefetch(2 * p + 2, 0)
            process(1)
            drain(1)
            prefetch(2 * p + 3, 1)
            process(0)
            return 0

        lax.fori_loop(0, (nch - 3) // 2, pair, 0)
        drain(0)
        prefetch(nch - 1, 0)
        process(1)
        drain(1)
        process(0)
        drain(0)
        plsc.subcore_barrier()
        pltpu.sync_copy(acc.at[pl.ds(s * rpt, rpt)],
                        out_h.at[c, pl.ds(s * rpt, rpt)])

    return k(hs, sd, ewx)


def _tc_stage1(x_p, deg0, deg1, w1):
    npad = x_p.shape[0]
    grid = (npad // _BR,)

    def body(xb, d0, d1, w, hb, hsb, dvb):
        dv = lax.rsqrt(d0[...] + d1[...] + 1.0)
        h = jnp.dot(xb[...], w[...], preferred_element_type=jnp.float32)
        hb[...] = h
        hsb[...] = h * dv
        dvb[...] = dv

    return pl.pallas_call(
        body,
        grid=grid,
        in_specs=[
            pl.BlockSpec((_BR, 128), lambda i: (i, 0)),
            pl.BlockSpec((_BR, 1), lambda i: (i, 0)),
            pl.BlockSpec((_BR, 1), lambda i: (i, 0)),
            pl.BlockSpec((128, 128), lambda i: (0, 0)),
        ],
        out_specs=[
            pl.BlockSpec((_BR, 128), lambda i: (i, 0)),
            pl.BlockSpec((_BR, 128), lambda i: (i, 0)),
            pl.BlockSpec((_BR, 1), lambda i: (i, 0)),
        ],
        out_shape=[
            jax.ShapeDtypeStruct((npad, 128), jnp.float32),
            jax.ShapeDtypeStruct((npad, 128), jnp.float32),
            jax.ShapeDtypeStruct((npad, 1), jnp.float32),
        ],
    )(x_p, deg0, deg1, w1)


def _tc_stage2(a0, a1, h1, dinv, b1, w2):
    npad = h1.shape[0]
    grid = (npad // _BR,)

    def body(a0b, a1b, hb, dvb, bb, wb, h2o, hs2o):
        dv = dvb[...]
        u = jnp.maximum(
            dv * (a0b[...] + a1b[...]) + dv * dv * hb[...] + bb[...], 0.0)
        h2 = jnp.dot(u, wb[...], preferred_element_type=jnp.float32)
        h2o[...] = h2
        hs2o[...] = h2 * dv

    return pl.pallas_call(
        body,
        grid=grid,
        in_specs=[
            pl.BlockSpec((_BR, 128), lambda i: (i, 0)),
            pl.BlockSpec((_BR, 128), lambda i: (i, 0)),
            pl.BlockSpec((_BR, 128), lambda i: (i, 0)),
            pl.BlockSpec((_BR, 1), lambda i: (i, 0)),
            pl.BlockSpec((1, 128), lambda i: (0, 0)),
            pl.BlockSpec((128, 128), lambda i: (0, 0)),
        ],
        out_specs=[
            pl.BlockSpec((_BR, 128), lambda i: (i, 0)),
            pl.BlockSpec((_BR, 128), lambda i: (i, 0)),
        ],
        out_shape=[
            jax.ShapeDtypeStruct((npad, 128), jnp.float32),
            jax.ShapeDtypeStruct((npad, 128), jnp.float32),
        ],
    )(a0, a1, h1, dinv, b1, w2)


def _tc_stage3(a0, a1, h2, dinv, b2, batch_row, wc, bc):
    npad = h2.shape[0]
    grid = (npad // _BR,)
    nc = wc.shape[1]

    def body(a0b, a1b, hb, dvb, bb, btb, wcb, bcb, out, pacc):
        i = pl.program_id(0)
        dv = dvb[...]
        v = jnp.maximum(
            dv * (a0b[...] + a1b[...]) + dv * dv * hb[...] + bb[...], 0.0)
        mt = (btb[...] == lax.broadcasted_iota(jnp.int32, (_G, _BR), 0))
        pb = lax.dot_general(mt.astype(jnp.float32), v,
                             (((1,), (0,)), ((), ())),
                             preferred_element_type=jnp.float32)

        @pl.when(i == 0)
        def _():
            pacc[...] = pb

        @pl.when(i > 0)
        def _():
            pacc[...] = pacc[...] + pb

        @pl.when(i == pl.num_programs(0) - 1)
        def _():
            logits = jnp.dot(pacc[...], wcb[...],
                             preferred_element_type=jnp.float32) + bcb[...]
            out[...] = jax.nn.sigmoid(logits)

    return pl.pallas_call(
        body,
        grid=grid,
        in_specs=[
            pl.BlockSpec((_BR, 128), lambda i: (i, 0)),
            pl.BlockSpec((_BR, 128), lambda i: (i, 0)),
            pl.BlockSpec((_BR, 128), lambda i: (i, 0)),
            pl.BlockSpec((_BR, 1), lambda i: (i, 0)),
            pl.BlockSpec((1, 128), lambda i: (0, 0)),
            pl.BlockSpec((1, _BR), lambda i: (0, i)),
            pl.BlockSpec((128, nc), lambda i: (0, 0)),
            pl.BlockSpec((1, nc), lambda i: (0, 0)),
        ],
        out_specs=pl.BlockSpec((_G, nc), lambda i: (0, 0)),
        out_shape=jax.ShapeDtypeStruct((_G, nc), jnp.float32),
        scratch_shapes=[pltpu.VMEM((_G, 128), jnp.float32)],
    )(a0, a1, h2, dinv, b2, batch_row, wc, bc)


def kernel(x, edge_index, edge_attr, batch, W1, b1, W2, b2, Wc, bc):
    n, d = x.shape
    npad = ((n + _BR - 1) // _BR) * _BR

    x_p = jnp.pad(x, ((0, npad - n), (0, 0)))
    batch_row = jnp.pad(batch, (0, npad - n),
                        constant_values=_G).reshape(1, npad)
    e = edge_attr.shape[0]
    sd = jnp.stack([edge_index[0].reshape(e // _CH, _CH),
                    edge_index[1].reshape(e // _CH, _CH)], axis=1)
    ewx = jnp.broadcast_to(edge_attr[:, None], (e, 16))

    degs = _deg_kernel(sd, ewx, npad)
    deg0 = lax.slice(degs, (0, 0, 0), (1, npad, 1)).reshape(npad, 1)
    deg1 = lax.slice(degs, (1, 0, 0), (2, npad, 1)).reshape(npad, 1)

    h1, hs1, dinv = _tc_stage1(x_p, deg0, deg1, W1)

    agg1 = _agg_kernel(hs1, sd, ewx, npad)
    h2, hs2 = _tc_stage2(agg1[0], agg1[1], h1, dinv,
                         b1.reshape(1, -1), W2)

    agg2 = _agg_kernel(hs2, sd, ewx, npad)
    return _tc_stage3(agg2[0], agg2[1], h2, dinv,
                      b2.reshape(1, -1), batch_row, Wc, bc.reshape(1, -1))
